# Initial kernel scaffold; baseline (speedup 1.0000x reference)
import jax
import jax.numpy as jnp
from jax.experimental import pallas as pl


def kernel(x_querry, x_block, e_p_0, e_k_0, e_a_0, l):
    raise NotImplementedError("write your pallas kernel here")


# speedup vs baseline: 1.0854x; 1.0854x over previous
import jax
import jax.numpy as jnp
from jax.experimental import pallas as pl
from jax.experimental.pallas import tpu as pltpu

_B = 128
_EMB = 768
_POOL = 100
_PLEN = 8
_HALF = _PLEN // 2
_EPS = 1e-6


def _body(gate_ref, x_ref, ea_ref, ek_ref, ep_ref, eko_ref, evo_ref):
    ea = ea_ref[...]
    m = jnp.max(ea, axis=1, keepdims=True)
    p = jnp.exp(ea - m)
    A = p / jnp.sum(p, axis=1, keepdims=True)

    ek = ek_ref[...]
    n2 = jnp.sqrt(jnp.sum(ek * ek, axis=1, keepdims=True))
    Wn = (A * ek) / jnp.maximum(n2, _EPS)

    x = x_ref[...]
    dn_t = (((1,), (1,)), ((), ()))
    num = jax.lax.dot_general(x, Wn, dn_t, preferred_element_type=jnp.float32)
    n1sq = jax.lax.dot_general(x * x, A * A, dn_t,
                               preferred_element_type=jnp.float32)
    n1 = jnp.maximum(jnp.sqrt(n1sq), _EPS)

    gate = gate_ref[0]
    aq = ((num / n1) + 1.0) * (0.5 * gate)

    dn = (((1,), (0,)), ((), ()))
    for l in range(_PLEN):
        dst = eko_ref if l < _HALF else evo_ref
        j = l if l < _HALF else l - _HALF
        dst[:, j * _EMB:(j + 1) * _EMB] = jax.lax.dot_general(
            aq, ep_ref[l], dn, preferred_element_type=jnp.float32)


def kernel(x_querry, x_block, e_p_0, e_k_0, e_a_0, l):
    in_layers = jnp.any(jnp.asarray(l) == jnp.asarray([0, 1, 2, 3, 4, 5]))
    gate = in_layers.astype(jnp.float32).reshape(1)

    out_t = (
        jax.ShapeDtypeStruct((_B, _HALF * _EMB), jnp.float32),
        jax.ShapeDtypeStruct((_B, _HALF * _EMB), jnp.float32),
    )
    ek2, ev2 = pl.pallas_call(
        _body,
        out_shape=out_t,
        in_specs=[
            pl.BlockSpec(memory_space=pltpu.SMEM),
            pl.BlockSpec(memory_space=pltpu.VMEM),
            pl.BlockSpec(memory_space=pltpu.VMEM),
            pl.BlockSpec(memory_space=pltpu.VMEM),
            pl.BlockSpec(memory_space=pltpu.VMEM),
        ],
        out_specs=(
            pl.BlockSpec(memory_space=pltpu.VMEM),
            pl.BlockSpec(memory_space=pltpu.VMEM),
        ),
    )(gate, x_querry, e_a_0, e_k_0, e_p_0)

    Ek = ek2.reshape(_B, _HALF, _EMB)
    Ev = ev2.reshape(_B, _HALF, _EMB)
    return (Ek, Ev, x_block)
